# baseline (speedup 1.0000x reference)
import functools

import jax
import jax.numpy as jnp
from jax import lax
from jax.experimental import pallas as pl
from jax.experimental.pallas import tpu as pltpu
from jax.experimental.pallas import tpu_sc as plsc

_NC = 2
_NS = 16
_NW = _NC * _NS
_LANES = 16


@functools.cache
def _make_kernel(batch, seq, emb):
    rows_per_w = batch // _NW
    assert rows_per_w % 2 == 0 and rows_per_w >= 6
    chunk_a = 104
    chunk_b = seq - chunk_a
    mesh = plsc.VectorSubcoreMesh(core_axis_name="c", subcore_axis_name="s")

    @functools.partial(
        pl.kernel,
        out_type=jax.ShapeDtypeStruct((batch, seq, emb), jnp.float32),
        mesh=mesh,
        compiler_params=pltpu.CompilerParams(use_tc_tiling_on_sc=True),
        scratch_types=[
            pltpu.VMEM((seq * emb,), jnp.float32),
            pltpu.VMEM((seq,), jnp.int32),
            pltpu.VMEM((seq,), jnp.int32),
            pltpu.VMEM((seq, 128), jnp.float32),
            pltpu.VMEM((seq, 128), jnp.float32),
            pltpu.VMEM((seq, emb), jnp.float32),
            pltpu.VMEM((seq, emb), jnp.float32),
            pltpu.SemaphoreType.DMA,
            pltpu.SemaphoreType.DMA,
            pltpu.SemaphoreType.DMA,
            pltpu.SemaphoreType.DMA,
            pltpu.SemaphoreType.DMA,
            pltpu.SemaphoreType.DMA,
        ],
    )
    def emb_kernel(idx_hbm, tok_hbm, pos_hbm, out_hbm,
                   pos_v, idx0, idx1, rows0, rows1, st0, st1,
                   is0, is1, in0, in1, os0, os1):
        wid = lax.axis_index("s") * _NC + lax.axis_index("c")
        base = wid * rows_per_w
        pltpu.sync_copy(pos_hbm, pos_v)

        idxs = (idx0, idx1)
        rows = (rows0, rows1)
        staged = (st0, st1)
        isems = (is0, is1)
        gsems = (in0, in1)
        osems = (os0, os1)

        def idx_cp(r_local, buf):
            return (idx_hbm.at[pl.ds((base + r_local) * seq, seq)],
                    idxs[buf], isems[buf])

        def gather_cps(buf):
            return [
                (tok_hbm.at[idxs[buf].at[pl.ds(0, chunk_a)]],
                 rows[buf].at[pl.ds(0, chunk_a)],
                 gsems[buf]),
                (tok_hbm.at[idxs[buf].at[pl.ds(chunk_a, chunk_b)]],
                 rows[buf].at[pl.ds(chunk_a, chunk_b)],
                 gsems[buf]),
            ]

        def out_cp(r_local, buf):
            return (staged[buf], out_hbm.at[base + r_local], osems[buf])

        def start(args):
            pltpu.async_copy(*args)

        def wait(args):
            pltpu.make_async_copy(*args).wait()

        def add_pos(buf):
            rv = rows[buf]
            sv = staged[buf]

            def body(i, c):
                for k in range(emb // _LANES):
                    sl = pl.ds(k * _LANES, _LANES)
                    sv[i, sl] = rv[i, sl] + pos_v[pl.ds(i * emb + k * _LANES,
                                                        _LANES)]
                return c

            lax.fori_loop(0, seq, body, 0)

        def iteration(r, b, *, warm_out, feed_gather, feed_idx):
            b2 = 1 - b
            if feed_gather:
                wait(idx_cp(r + 1, b2))
                for args in gather_cps(b2):
                    start(args)
            for args in gather_cps(b):
                wait(args)
            if feed_idx:
                start(idx_cp(r + 2, b))
            if warm_out:
                wait(out_cp(r - 2, b))
            add_pos(b)
            start(out_cp(r, b))

        start(idx_cp(0, 0))
        start(idx_cp(1, 1))
        wait(idx_cp(0, 0))
        for args in gather_cps(0):
            start(args)

        iteration(0, 0, warm_out=False, feed_gather=True, feed_idx=True)
        iteration(1, 1, warm_out=False, feed_gather=True, feed_idx=True)

        def pair(k, c):
            r = 2 * k + 2
            iteration(r, 0, warm_out=True, feed_gather=True, feed_idx=True)
            iteration(r + 1, 1, warm_out=True, feed_gather=True, feed_idx=True)
            return c

        lax.fori_loop(0, (rows_per_w - 4) // 2, pair, 0)

        iteration(rows_per_w - 2, 0, warm_out=True, feed_gather=True,
                  feed_idx=False)
        iteration(rows_per_w - 1, 1, warm_out=True, feed_gather=False,
                  feed_idx=False)
        wait(out_cp(rows_per_w - 2, 0))
        wait(out_cp(rows_per_w - 1, 1))

    return emb_kernel


def kernel(inputs, token_table, position_table):
    batch, seq = inputs.shape
    emb = token_table.shape[1]
    idx = inputs.astype(jnp.int32).reshape(batch * seq)
    tok128 = jnp.pad(token_table, ((0, 0), (0, 128 - emb)))
    pos_flat = position_table.reshape(seq * emb)
    half = batch // 2
    f = _make_kernel(half, seq, emb)
    parts = [f(idx[i * half * seq:(i + 1) * half * seq], tok128, pos_flat)
             for i in range(2)]
    return jnp.concatenate(parts, axis=0)


# speedup vs baseline: 1.2065x; 1.2065x over previous
import functools

import jax
import jax.numpy as jnp
from jax import lax
from jax.experimental import pallas as pl
from jax.experimental.pallas import tpu as pltpu
from jax.experimental.pallas import tpu_sc as plsc

_NC = 2
_NS = 16
_NW = _NC * _NS
_LANES = 16


@functools.cache
def _make_kernel(batch, seq, emb):
    rows_per_w = batch // _NW
    assert rows_per_w % 2 == 0 and rows_per_w >= 6
    chunk_a = 104
    chunk_b = seq - chunk_a
    mesh = plsc.VectorSubcoreMesh(core_axis_name="c", subcore_axis_name="s")

    @functools.partial(
        pl.kernel,
        out_type=jax.ShapeDtypeStruct((batch, seq, emb), jnp.float32),
        mesh=mesh,
        compiler_params=pltpu.CompilerParams(use_tc_tiling_on_sc=True),
        scratch_types=[
            pltpu.VMEM((seq * emb,), jnp.float32),
            pltpu.VMEM((seq,), jnp.int32),
            pltpu.VMEM((seq,), jnp.int32),
            pltpu.VMEM((seq, 128), jnp.float32),
            pltpu.VMEM((seq, 128), jnp.float32),
            pltpu.VMEM((seq, emb), jnp.float32),
            pltpu.VMEM((seq, emb), jnp.float32),
            pltpu.SemaphoreType.DMA,
            pltpu.SemaphoreType.DMA,
            pltpu.SemaphoreType.DMA,
            pltpu.SemaphoreType.DMA,
            pltpu.SemaphoreType.DMA,
            pltpu.SemaphoreType.DMA,
        ],
    )
    def emb_kernel(idx_hbm, tok_hbm, pos_hbm, out_hbm,
                   pos_v, idx0, idx1, rows0, rows1, st0, st1,
                   is0, is1, in0, in1, os0, os1):
        wid = lax.axis_index("s") * _NC + lax.axis_index("c")
        base = wid * rows_per_w
        pltpu.sync_copy(pos_hbm, pos_v)

        idxs = (idx0, idx1)
        rows = (rows0, rows1)
        staged = (st0, st1)
        isems = (is0, is1)
        gsems = (in0, in1)
        osems = (os0, os1)

        def idx_cp(r_local, buf):
            return (idx_hbm.at[pl.ds((base + r_local) * seq, seq)],
                    idxs[buf], isems[buf])

        def gather_cps(buf):
            return [
                (tok_hbm.at[idxs[buf].at[pl.ds(0, chunk_a)]],
                 rows[buf].at[pl.ds(0, chunk_a)],
                 gsems[buf]),
                (tok_hbm.at[idxs[buf].at[pl.ds(chunk_a, chunk_b)]],
                 rows[buf].at[pl.ds(chunk_a, chunk_b)],
                 gsems[buf]),
            ]

        def out_cp(r_local, buf):
            return (staged[buf], out_hbm.at[base + r_local], osems[buf])

        def start(args):
            pltpu.async_copy(*args)

        def wait(args):
            pltpu.make_async_copy(*args).wait()

        def add_pos(buf):
            rv = rows[buf]
            sv = staged[buf]

            def body(i, c):
                for k in range(emb // _LANES):
                    sl = pl.ds(k * _LANES, _LANES)
                    sv[i, sl] = rv[i, sl] + pos_v[pl.ds(i * emb + k * _LANES,
                                                        _LANES)]
                return c

            lax.fori_loop(0, seq, body, 0)

        def iteration(r, b, *, warm_out, feed_gather, feed_idx):
            b2 = 1 - b
            if feed_gather:
                wait(idx_cp(r + 1, b2))
                for args in gather_cps(b2):
                    start(args)
            for args in gather_cps(b):
                wait(args)
            if feed_idx:
                start(idx_cp(r + 2, b))
            if warm_out:
                wait(out_cp(r - 2, b))
            add_pos(b)
            start(out_cp(r, b))

        start(idx_cp(0, 0))
        start(idx_cp(1, 1))
        wait(idx_cp(0, 0))
        for args in gather_cps(0):
            start(args)

        iteration(0, 0, warm_out=False, feed_gather=True, feed_idx=True)
        iteration(1, 1, warm_out=False, feed_gather=True, feed_idx=True)

        def pair(k, c):
            r = 2 * k + 2
            iteration(r, 0, warm_out=True, feed_gather=True, feed_idx=True)
            iteration(r + 1, 1, warm_out=True, feed_gather=True, feed_idx=True)
            return c

        lax.fori_loop(0, (rows_per_w - 4) // 2, pair, 0)

        iteration(rows_per_w - 2, 0, warm_out=True, feed_gather=True,
                  feed_idx=False)
        iteration(rows_per_w - 1, 1, warm_out=True, feed_gather=False,
                  feed_idx=False)
        wait(out_cp(rows_per_w - 2, 0))
        wait(out_cp(rows_per_w - 1, 1))

    return emb_kernel


def kernel(inputs, token_table, position_table):
    batch, seq = inputs.shape
    emb = token_table.shape[1]
    idx = inputs.astype(jnp.int32).reshape(batch * seq)
    tok128 = jnp.pad(token_table, ((0, 0), (0, 128 - emb)))
    pos_flat = position_table.reshape(seq * emb)
    f = _make_kernel(batch, seq, emb)
    return f(idx, tok128, pos_flat)


# speedup vs baseline: 1.5662x; 1.2982x over previous
import functools

import jax
import jax.numpy as jnp
from jax import lax
from jax.experimental import pallas as pl
from jax.experimental.pallas import tpu as pltpu
from jax.experimental.pallas import tpu_sc as plsc

_NC = 2
_NS = 16
_NW = _NC * _NS
_LANES = 16


@functools.cache
def _make_kernel(batch, seq, emb):
    rows_per_w = batch // _NW
    assert rows_per_w % 2 == 0 and rows_per_w >= 10
    chunk_a = 104
    chunk_b = seq - chunk_a
    mesh = plsc.VectorSubcoreMesh(core_axis_name="c", subcore_axis_name="s")

    @functools.partial(
        pl.kernel,
        out_type=jax.ShapeDtypeStruct((batch, seq, 128), jnp.float32),
        mesh=mesh,
        compiler_params=pltpu.CompilerParams(use_tc_tiling_on_sc=True),
        scratch_types=[
            pltpu.VMEM((seq * emb,), jnp.float32),
            pltpu.VMEM((seq,), jnp.int32),
            pltpu.VMEM((seq,), jnp.int32),
            pltpu.VMEM((seq, 128), jnp.float32),
            pltpu.VMEM((seq, 128), jnp.float32),
            pltpu.VMEM((seq, 128), jnp.float32),
            pltpu.SemaphoreType.DMA,
            pltpu.SemaphoreType.DMA,
            pltpu.SemaphoreType.DMA,
            pltpu.SemaphoreType.DMA,
            pltpu.SemaphoreType.DMA,
            pltpu.SemaphoreType.DMA,
            pltpu.SemaphoreType.DMA,
            pltpu.SemaphoreType.DMA,
        ],
    )
    def emb_kernel(idx_hbm, tok_hbm, pos_hbm, out_hbm,
                   pos_v, idx0, idx1, rows0, rows1, rows2,
                   is0, is1, g0, g1, g2, o0, o1, o2):
        wid = lax.axis_index("s") * _NC + lax.axis_index("c")
        base = wid * rows_per_w
        pltpu.sync_copy(pos_hbm, pos_v)

        idxs = (idx0, idx1)
        rows = (rows0, rows1, rows2)
        isems = (is0, is1)
        gsems = (g0, g1, g2)
        osems = (o0, o1, o2)

        def idx_cp(r_local, ib):
            return (idx_hbm.at[pl.ds((base + r_local) * seq, seq)],
                    idxs[ib], isems[ib])

        def gather_cps(ib, b):
            return [
                (tok_hbm.at[idxs[ib].at[pl.ds(0, chunk_a)]],
                 rows[b].at[pl.ds(0, chunk_a)],
                 gsems[b]),
                (tok_hbm.at[idxs[ib].at[pl.ds(chunk_a, chunk_b)]],
                 rows[b].at[pl.ds(chunk_a, chunk_b)],
                 gsems[b]),
            ]

        def out_cp(r_local, b):
            return (rows[b], out_hbm.at[base + r_local], osems[b])

        def start(args):
            pltpu.async_copy(*args)

        def wait(args):
            pltpu.make_async_copy(*args).wait()

        def add_pos(b):
            rv = rows[b]

            def body(i, c):
                for k in range(emb // _LANES):
                    sl = pl.ds(k * _LANES, _LANES)
                    plsc.addupdate(rv.at[i, sl],
                                   pos_v[pl.ds(i * emb + k * _LANES, _LANES)])
                return c

            lax.fori_loop(0, seq, body, 0)

        def iteration(r, b, ib, *, warm_out, feed_gather, feed_idx):
            if feed_gather:
                if warm_out:
                    wait(out_cp(r - 2, (b + 1) % 3))
                wait(idx_cp(r + 1, 1 - ib))
                for args in gather_cps(1 - ib, (b + 1) % 3):
                    start(args)
            for args in gather_cps(ib, b):
                wait(args)
            if feed_idx:
                start(idx_cp(r + 2, ib))
            add_pos(b)
            start(out_cp(r, b))

        start(idx_cp(0, 0))
        start(idx_cp(1, 1))
        wait(idx_cp(0, 0))
        for args in gather_cps(0, 0):
            start(args)

        iteration(0, 0, 0, warm_out=False, feed_gather=True, feed_idx=True)
        iteration(1, 1, 1, warm_out=False, feed_gather=True, feed_idx=True)
        for r in range(2, 8):
            iteration(r, r % 3, r % 2,
                      warm_out=True, feed_gather=True, feed_idx=True)

        assert (rows_per_w - 14) % 6 == 0
        def six(k, c):
            r = 6 * k + 8
            for j in range(6):
                iteration(r + j, (8 + j) % 3, j % 2,
                          warm_out=True, feed_gather=True, feed_idx=True)
            return c

        lax.fori_loop(0, (rows_per_w - 14) // 6, six, 0)

        rt = rows_per_w - 6
        for j in range(6):
            r = rt + j
            iteration(r, r % 3, r % 2, warm_out=True,
                      feed_gather=(j < 5), feed_idx=(j < 4))
        wait(out_cp(rows_per_w - 2, (rows_per_w - 2) % 3))
        wait(out_cp(rows_per_w - 1, (rows_per_w - 1) % 3))

    return emb_kernel


def kernel(inputs, token_table, position_table):
    batch, seq = inputs.shape
    emb = token_table.shape[1]
    idx = inputs.astype(jnp.int32).reshape(batch * seq)
    tok128 = jnp.pad(token_table, ((0, 0), (0, 128 - emb)))
    pos_flat = position_table.reshape(seq * emb)
    f = _make_kernel(batch, seq, emb)
    return f(idx, tok128, pos_flat)[:, :, :emb]


# speedup vs baseline: 1.5671x; 1.0006x over previous
import functools

import jax
import jax.numpy as jnp
from jax import lax
from jax.experimental import pallas as pl
from jax.experimental.pallas import tpu as pltpu
from jax.experimental.pallas import tpu_sc as plsc

_NC = 2
_NS = 16
_NW = _NC * _NS
_LANES = 16


@functools.cache
def _make_kernel(batch, seq, emb):
    rows_per_w = batch // _NW
    assert rows_per_w % 2 == 0 and rows_per_w >= 10
    chunk_a = 104
    chunk_b = seq - chunk_a
    mesh = plsc.VectorSubcoreMesh(core_axis_name="c", subcore_axis_name="s")

    @functools.partial(
        pl.kernel,
        out_type=jax.ShapeDtypeStruct((batch, seq, 128), jnp.float32),
        mesh=mesh,
        compiler_params=pltpu.CompilerParams(use_tc_tiling_on_sc=True),
        scratch_types=[
            pltpu.VMEM((seq * emb,), jnp.float32),
            pltpu.VMEM((seq,), jnp.int32),
            pltpu.VMEM((seq,), jnp.int32),
            pltpu.VMEM((seq, 128), jnp.float32),
            pltpu.VMEM((seq, 128), jnp.float32),
            pltpu.VMEM((seq, 128), jnp.float32),
            pltpu.VMEM((seq, 128), jnp.float32),
            pltpu.SemaphoreType.DMA,
            pltpu.SemaphoreType.DMA,
            pltpu.SemaphoreType.DMA,
            pltpu.SemaphoreType.DMA,
            pltpu.SemaphoreType.DMA,
            pltpu.SemaphoreType.DMA,
            pltpu.SemaphoreType.DMA,
            pltpu.SemaphoreType.DMA,
            pltpu.SemaphoreType.DMA,
            pltpu.SemaphoreType.DMA,
        ],
    )
    def emb_kernel(idx_hbm, tok_hbm, pos_hbm, out_hbm,
                   pos_v, idx0, idx1, rows0, rows1, rows2, rows3,
                   is0, is1, g0, g1, g2, g3, o0, o1, o2, o3):
        wid = lax.axis_index("s") * _NC + lax.axis_index("c")
        base = wid * rows_per_w
        pltpu.sync_copy(pos_hbm, pos_v)

        idxs = (idx0, idx1)
        rows = (rows0, rows1, rows2, rows3)
        isems = (is0, is1)
        gsems = (g0, g1, g2, g3)
        osems = (o0, o1, o2, o3)

        def idx_cp(r_local, ib):
            return (idx_hbm.at[pl.ds((base + r_local) * seq, seq)],
                    idxs[ib], isems[ib])

        def gather_cps(ib, b):
            return [
                (tok_hbm.at[idxs[ib].at[pl.ds(0, chunk_a)]],
                 rows[b].at[pl.ds(0, chunk_a)],
                 gsems[b]),
                (tok_hbm.at[idxs[ib].at[pl.ds(chunk_a, chunk_b)]],
                 rows[b].at[pl.ds(chunk_a, chunk_b)],
                 gsems[b]),
            ]

        def out_cp(r_local, b):
            return (rows[b], out_hbm.at[base + r_local], osems[b])

        def start(args):
            pltpu.async_copy(*args)

        def wait(args):
            pltpu.make_async_copy(*args).wait()

        def add_pos(b):
            rv = rows[b]

            def body(i, c):
                for k in range(emb // _LANES):
                    sl = pl.ds(k * _LANES, _LANES)
                    plsc.addupdate(rv.at[i, sl],
                                   pos_v[pl.ds(i * emb + k * _LANES, _LANES)])
                return c

            lax.fori_loop(0, seq, body, 0)

        def iteration(r, b, ib, *, warm_out, feed_gather, feed_idx):
            if feed_gather:
                if warm_out:
                    wait(out_cp(r - 3, (b + 1) % 4))
                wait(idx_cp(r + 1, 1 - ib))
                for args in gather_cps(1 - ib, (b + 1) % 4):
                    start(args)
            for args in gather_cps(ib, b):
                wait(args)
            if feed_idx:
                start(idx_cp(r + 2, ib))
            add_pos(b)
            start(out_cp(r, b))

        start(idx_cp(0, 0))
        start(idx_cp(1, 1))
        wait(idx_cp(0, 0))
        for args in gather_cps(0, 0):
            start(args)

        iteration(0, 0, 0, warm_out=False, feed_gather=True, feed_idx=True)
        iteration(1, 1, 1, warm_out=False, feed_gather=True, feed_idx=True)
        iteration(2, 2, 0, warm_out=False, feed_gather=True, feed_idx=True)
        iteration(3, 3, 1, warm_out=True, feed_gather=True, feed_idx=True)

        assert (rows_per_w - 8) % 4 == 0
        def quad(k, c):
            r = 4 * k + 4
            for j in range(4):
                iteration(r + j, j, j % 2,
                          warm_out=True, feed_gather=True, feed_idx=True)
            return c

        lax.fori_loop(0, (rows_per_w - 8) // 4, quad, 0)

        rt = rows_per_w - 4
        for j in range(4):
            r = rt + j
            iteration(r, r % 4, r % 2, warm_out=True,
                      feed_gather=(j < 3), feed_idx=(j < 2))
        for j in range(4):
            r = rows_per_w - 4 + j
            wait(out_cp(r, r % 4))

    return emb_kernel


def kernel(inputs, token_table, position_table):
    batch, seq = inputs.shape
    emb = token_table.shape[1]
    idx = inputs.astype(jnp.int32).reshape(batch * seq)
    tok128 = jnp.pad(token_table, ((0, 0), (0, 128 - emb)))
    pos_flat = position_table.reshape(seq * emb)
    f = _make_kernel(batch, seq, emb)
    return f(idx, tok128, pos_flat)[:, :, :emb]
